# baseline (speedup 1.0000x reference)
import jax
import jax.numpy as jnp
from jax import lax
from jax.experimental import pallas as pl
from jax.experimental.pallas import tpu as pltpu

N = 16384
D = 128
G = 100
NQ = 4
BLK = 4096
BLKQ = BLK // NQ
NB = N // BLK
NR = N // BLKQ
EPS = 1e-8
NEG = -1e30
HI_BITS = 0x40800000


def _body(x0, x1, x2, x3, lbl_ref, g_ref, loss_ref, keys_s, nll_s, accf, acci):
    i = pl.program_id(0)

    @pl.when(i == 0)
    def _init():
        accf[0] = 0.0
        acci[0] = 0
        acci[1] = 0

    g = g_ref[:, :]

    row = lax.broadcasted_iota(jnp.int32, (D, BLKQ), 0)

    gss = jnp.sum(g * g, axis=1, keepdims=True)
    gn = g / jnp.maximum(jnp.sqrt(gss), EPS)

    eye = (lax.broadcasted_iota(jnp.int32, (D, D), 0)
           == lax.broadcasted_iota(jnp.int32, (D, D), 1)).astype(jnp.float32)
    ones_r = jnp.ones((1, D), jnp.float32)
    dn = (((1,), (1,)), ((), ()))
    dn0 = (((1,), (0,)), ((), ()))

    sum_p = jnp.zeros((1, 1), jnp.float32)
    cnt_p = jnp.zeros((1, 1), jnp.int32)
    cnt_u = jnp.zeros((1, 1), jnp.int32)

    for q, x_ref in enumerate((x0, x1, x2, x3)):
        x = x_ref[:, :]
        xT = lax.dot_general(eye, x, dn, preferred_element_type=jnp.float32)
        simsT = lax.dot_general(gn, x, dn, preferred_element_type=jnp.float32)

        xss = lax.dot_general(ones_r, xT * xT, dn0,
                              preferred_element_type=jnp.float32)
        rinv = 1.0 / jnp.maximum(jnp.sqrt(xss), EPS)
        simsm = jnp.where(row < G, simsT * rinv, NEG)

        top1 = jnp.max(simsm, axis=0, keepdims=True)
        arg1 = jnp.min(jnp.where(simsm == top1, row, D), axis=0, keepdims=True)
        top2 = jnp.max(jnp.where(row == arg1, NEG, simsm), axis=0,
                       keepdims=True)
        diff = top1 - top2

        m = jnp.max(xT, axis=0, keepdims=True)
        esum = lax.dot_general(ones_r, jnp.exp(xT - m), dn0,
                               preferred_element_type=jnp.float32)
        lse = m + jnp.log(esum)

        lbl = lbl_ref[q].astype(jnp.int32)
        p_mask = lbl <= G - 1
        tgt = jnp.where(p_mask, lbl, arg1)
        tval = lax.dot_general(ones_r, jnp.where(row == tgt, xT, 0.0), dn0,
                               preferred_element_type=jnp.float32)
        nll = lse - tval

        u_mask = jnp.logical_not(p_mask)
        keys = jnp.where(u_mask, lax.bitcast_convert_type(diff, jnp.int32),
                         jnp.int32(-1))

        keys_s[pl.ds(i * NQ + q, 1), :] = keys
        nll_s[pl.ds(i * NQ + q, 1), :] = jnp.where(u_mask, nll, 0.0)

        sum_p += jnp.sum(jnp.where(p_mask, nll, 0.0), keepdims=True)
        cnt_p += jnp.sum(p_mask.astype(jnp.int32), keepdims=True)
        cnt_u += jnp.sum(u_mask.astype(jnp.int32), keepdims=True)

    accf[0] += sum_p[0, 0]
    acci[0] += cnt_p[0, 0]
    acci[1] += cnt_u[0, 0]

    @pl.when(i == NB - 1)
    def _finish():
        k = acci[1] // 10
        keys_all = keys_s[:, :]
        nll_all = nll_s[:, :]

        def bs(_, c):
            lo, hi = c
            mid = lo + (hi - lo) // 2
            cnt = jnp.sum((keys_all >= mid).astype(jnp.int32), keepdims=True)
            ok = cnt >= k
            return jnp.where(ok, mid, lo), jnp.where(ok, hi, mid)

        tau, _ = lax.fori_loop(
            0, 31, bs,
            (jnp.zeros((1, 1), jnp.int32), jnp.full((1, 1), HI_BITS, jnp.int32)))

        gt = keys_all > tau
        cgt = jnp.sum(gt.astype(jnp.int32), keepdims=True)
        need = k - cgt
        sum_gt = jnp.sum(jnp.where(gt, nll_all, 0.0), keepdims=True)

        tie = keys_all == tau
        idx = (lax.broadcasted_iota(jnp.int32, (NR, BLKQ), 0) * BLKQ
               + lax.broadcasted_iota(jnp.int32, (NR, BLKQ), 1))

        def bs2(_, c):
            lo, hi = c
            mid = lo + (hi - lo + 1) // 2
            cnt = jnp.sum((tie & (idx < mid)).astype(jnp.int32), keepdims=True)
            ok = cnt <= need
            return jnp.where(ok, mid, lo), jnp.where(ok, hi, mid - 1)

        mcut, _ = lax.fori_loop(
            0, 15, bs2,
            (jnp.zeros((1, 1), jnp.int32), jnp.full((1, 1), N, jnp.int32)))
        sum_tie = jnp.sum(jnp.where(tie & (idx < mcut), nll_all, 0.0),
                          keepdims=True)

        total = accf[0] + sum_gt + sum_tie
        cnt_all = acci[0] + k
        loss_ref[:, :] = total / cnt_all.astype(jnp.float32)


@jax.jit
def _run(outputs, labels3, gpad):
    xspec = [
        pl.BlockSpec((BLKQ, D), lambda i, q=q: (i * NQ + q, 0))
        for q in range(NQ)
    ]
    return pl.pallas_call(
        _body,
        grid=(NB,),
        in_specs=xspec + [
            pl.BlockSpec((NQ, 1, BLKQ), lambda i: (i, 0, 0)),
            pl.BlockSpec((D, D), lambda i: (0, 0)),
        ],
        out_specs=pl.BlockSpec((1, 1), lambda i: (0, 0)),
        out_shape=jax.ShapeDtypeStruct((1, 1), jnp.float32),
        scratch_shapes=[
            pltpu.VMEM((NR, BLKQ), jnp.int32),
            pltpu.VMEM((NR, BLKQ), jnp.float32),
            pltpu.SMEM((1,), jnp.float32),
            pltpu.SMEM((2,), jnp.int32),
        ],
        compiler_params=pltpu.CompilerParams(
            dimension_semantics=("arbitrary",),
        ),
    )(outputs, outputs, outputs, outputs, labels3, gpad)


def kernel(outputs, labels, global_logit):
    outputs = outputs.astype(jnp.float32)
    labels3 = labels.astype(jnp.int32).reshape(NR, 1, BLKQ)
    gpad = jnp.zeros((D, D), jnp.float32).at[:G].set(global_logit)
    return _run(outputs, labels3, gpad)[0, 0]


# speedup vs baseline: 1.3832x; 1.3832x over previous
import jax
import jax.numpy as jnp
from jax import lax
from jax.experimental import pallas as pl
from jax.experimental.pallas import tpu as pltpu

N = 16384
D = 128
G = 100
BLK = 4096
NB = N // BLK
EPS = 1e-8
NEG = -1e30
HI_BITS = 0x40000001


def _body(out_ref, lbl_ref, g_ref, loss_ref, keys_s, nll_s, accf, acci):
    i = pl.program_id(0)

    @pl.when(i == 0)
    def _init():
        accf[0] = 0.0
        acci[0] = 0
        acci[1] = 0

    x = out_ref[:, :]
    g = g_ref[:, :]

    row = lax.broadcasted_iota(jnp.int32, (D, BLK), 0)

    gss = jnp.sum(g * g, axis=1, keepdims=True)
    gn = g / jnp.maximum(jnp.sqrt(gss), EPS)

    eye = (lax.broadcasted_iota(jnp.int32, (D, D), 0)
           == lax.broadcasted_iota(jnp.int32, (D, D), 1)).astype(jnp.float32)
    ones_r = jnp.ones((1, D), jnp.float32)
    dn = (((1,), (1,)), ((), ()))
    dn0 = (((1,), (0,)), ((), ()))
    xT = lax.dot_general(eye, x, dn, preferred_element_type=jnp.float32)
    simsT = lax.dot_general(gn, x, dn, preferred_element_type=jnp.float32)

    xss = lax.dot_general(ones_r, xT * xT, dn0,
                          preferred_element_type=jnp.float32)
    rinv = 1.0 / jnp.maximum(jnp.sqrt(xss), EPS)
    simsm = jnp.where(row < G, simsT * rinv, NEG)

    top1 = jnp.max(simsm, axis=0, keepdims=True)
    arg1 = jnp.min(jnp.where(simsm == top1, row, D), axis=0, keepdims=True)
    top2 = jnp.max(jnp.where(row == arg1, NEG, simsm), axis=0, keepdims=True)
    diff = top1 - top2

    m = jnp.max(xT, axis=0, keepdims=True)
    esum = lax.dot_general(ones_r, jnp.exp(xT - m), dn0,
                           preferred_element_type=jnp.float32)
    lse = m + jnp.log(esum)

    lbl = lbl_ref[0].astype(jnp.int32)
    p_mask = lbl <= G - 1
    tgt = jnp.where(p_mask, lbl, arg1)
    tval = lax.dot_general(ones_r, jnp.where(row == tgt, xT, 0.0), dn0,
                           preferred_element_type=jnp.float32)
    nll = lse - tval

    u_mask = jnp.logical_not(p_mask)
    keys = jnp.where(u_mask, lax.bitcast_convert_type(diff, jnp.int32),
                     jnp.int32(-1))

    keys_s[pl.ds(i, 1), :] = keys
    nll_s[pl.ds(i, 1), :] = jnp.where(u_mask, nll, 0.0)

    accf[0] += jnp.sum(jnp.where(p_mask, nll, 0.0))
    acci[0] += jnp.sum(p_mask.astype(jnp.int32))
    acci[1] += jnp.sum(u_mask.astype(jnp.int32))

    @pl.when(i == NB - 1)
    def _finish():
        k = acci[1] // 10
        keys_all = keys_s[:, :]
        nll_all = nll_s[:, :]

        def bs(_, c):
            lo, hi = c
            qq = jnp.maximum((hi - lo) // 4, 1)
            m1 = lo + qq
            m2 = lo + 2 * qq
            m3 = lo + 3 * qq
            c1 = jnp.sum((keys_all >= m1).astype(jnp.int32))
            c2 = jnp.sum((keys_all >= m2).astype(jnp.int32))
            c3 = jnp.sum((keys_all >= m3).astype(jnp.int32))
            ok1 = c1 >= k
            ok2 = c2 >= k
            ok3 = c3 >= k
            lo_n = jnp.where(ok3, m3, jnp.where(ok2, m2, jnp.where(ok1, m1, lo)))
            hi_n = jnp.where(ok3, hi, jnp.where(ok2, m3, jnp.where(ok1, m2, m1)))
            return lo_n, hi_n

        tau, _ = lax.fori_loop(0, 18, bs,
                               (jnp.int32(0), jnp.int32(HI_BITS)))

        gt = keys_all > tau
        cgt = jnp.sum(gt.astype(jnp.int32))
        need = k - cgt
        sum_gt = jnp.sum(jnp.where(gt, nll_all, 0.0))

        tie = keys_all == tau
        tie_total = jnp.sum(tie.astype(jnp.int32))
        nll_tie = jnp.where(tie, nll_all, 0.0)

        def _tie_all():
            return jnp.sum(nll_tie)

        def _tie_search():
            idx = (lax.broadcasted_iota(jnp.int32, (NB, BLK), 0) * BLK
                   + lax.broadcasted_iota(jnp.int32, (NB, BLK), 1))

            def bs2(_, c):
                lo, hi = c
                mid = lo + (hi - lo + 1) // 2
                cnt = jnp.sum((tie & (idx < mid)).astype(jnp.int32))
                ok = cnt <= need
                return jnp.where(ok, mid, lo), jnp.where(ok, hi, mid - 1)

            mcut, _ = lax.fori_loop(0, 15, bs2, (jnp.int32(0), jnp.int32(N)))
            return jnp.sum(jnp.where(idx < mcut, nll_tie, 0.0))

        sum_tie = lax.cond(tie_total == need, _tie_all, _tie_search)

        total = accf[0] + sum_gt + sum_tie
        cnt_all = acci[0] + k
        loss_ref[:, :] = jnp.broadcast_to(total / cnt_all.astype(jnp.float32),
                                          (1, 1))


@jax.jit
def _run(outputs, labels3, gpad):
    return pl.pallas_call(
        _body,
        grid=(NB,),
        in_specs=[
            pl.BlockSpec((BLK, D), lambda i: (i, 0)),
            pl.BlockSpec((1, 1, BLK), lambda i: (i, 0, 0)),
            pl.BlockSpec((D, D), lambda i: (0, 0)),
        ],
        out_specs=pl.BlockSpec((1, 1), lambda i: (0, 0)),
        out_shape=jax.ShapeDtypeStruct((1, 1), jnp.float32),
        scratch_shapes=[
            pltpu.VMEM((NB, BLK), jnp.int32),
            pltpu.VMEM((NB, BLK), jnp.float32),
            pltpu.SMEM((1,), jnp.float32),
            pltpu.SMEM((2,), jnp.int32),
        ],
        compiler_params=pltpu.CompilerParams(
            dimension_semantics=("arbitrary",),
        ),
    )(outputs, labels3, gpad)


def kernel(outputs, labels, global_logit):
    outputs = outputs.astype(jnp.float32)
    labels3 = labels.astype(jnp.int32).reshape(NB, 1, BLK)
    gpad = jnp.zeros((D, D), jnp.float32).at[:G].set(global_logit)
    return _run(outputs, labels3, gpad)[0, 0]


# speedup vs baseline: 1.3994x; 1.0118x over previous
import jax
import jax.numpy as jnp
from jax import lax
from jax.experimental import pallas as pl
from jax.experimental.pallas import tpu as pltpu

N = 16384
D = 128
G = 100
BLK = 4096
NB = N // BLK
EPS = 1e-8
NEG = -1e30
HI_BITS = 0x40000001


def _body(out_ref, lbl_ref, g_ref, loss_ref, keys_s, nll_s, accf, acci):
    i = pl.program_id(0)

    @pl.when(i == 0)
    def _init():
        accf[0] = 0.0
        acci[0] = 0
        acci[1] = 0

    x = out_ref[:, :]
    g = g_ref[:, :]

    row = lax.broadcasted_iota(jnp.int32, (D, BLK), 0)

    gss = jnp.sum(g * g, axis=1, keepdims=True)
    gn = g / jnp.maximum(jnp.sqrt(gss), EPS)

    ones_r = jnp.ones((1, D), jnp.float32)
    dn = (((1,), (1,)), ((), ()))
    dn0 = (((1,), (0,)), ((), ()))
    xT = jnp.transpose(x)
    simsT = lax.dot_general(gn, x, dn, preferred_element_type=jnp.float32)

    xss = lax.dot_general(ones_r, xT * xT, dn0,
                          preferred_element_type=jnp.float32)
    rinv = 1.0 / jnp.maximum(jnp.sqrt(xss), EPS)
    simsm = jnp.where(row < G, simsT * rinv, NEG)

    top1 = jnp.max(simsm, axis=0, keepdims=True)
    arg1 = jnp.min(jnp.where(simsm == top1, row, D), axis=0, keepdims=True)
    top2 = jnp.max(jnp.where(row == arg1, NEG, simsm), axis=0, keepdims=True)
    diff = top1 - top2

    m = jnp.max(xT, axis=0, keepdims=True)
    esum = lax.dot_general(ones_r, jnp.exp(xT - m), dn0,
                           preferred_element_type=jnp.float32)
    lse = m + jnp.log(esum)

    lbl = lbl_ref[0].astype(jnp.int32)
    p_mask = lbl <= G - 1
    tgt = jnp.where(p_mask, lbl, arg1)
    tval = lax.dot_general(ones_r, jnp.where(row == tgt, xT, 0.0), dn0,
                           preferred_element_type=jnp.float32)
    nll = lse - tval

    u_mask = jnp.logical_not(p_mask)
    keys = jnp.where(u_mask, lax.bitcast_convert_type(diff, jnp.int32),
                     jnp.int32(-1))

    keys_s[pl.ds(i, 1), :] = keys
    nll_s[pl.ds(i, 1), :] = jnp.where(u_mask, nll, 0.0)

    accf[0] += jnp.sum(jnp.where(p_mask, nll, 0.0))
    acci[0] += jnp.sum(p_mask.astype(jnp.int32))
    acci[1] += jnp.sum(u_mask.astype(jnp.int32))

    @pl.when(i == NB - 1)
    def _finish():
        k = acci[1] // 10
        keys_all = keys_s[:, :]
        nll_all = nll_s[:, :]

        def bs(_, c):
            lo, hi = c
            qq = jnp.maximum((hi - lo) // 4, 1)
            m1 = lo + qq
            m2 = lo + 2 * qq
            m3 = lo + 3 * qq
            c1 = jnp.sum((keys_all >= m1).astype(jnp.int32))
            c2 = jnp.sum((keys_all >= m2).astype(jnp.int32))
            c3 = jnp.sum((keys_all >= m3).astype(jnp.int32))
            ok1 = c1 >= k
            ok2 = c2 >= k
            ok3 = c3 >= k
            lo_n = jnp.where(ok3, m3, jnp.where(ok2, m2, jnp.where(ok1, m1, lo)))
            hi_n = jnp.where(ok3, hi, jnp.where(ok2, m3, jnp.where(ok1, m2, m1)))
            return lo_n, hi_n

        tau, _ = lax.fori_loop(0, 18, bs,
                               (jnp.int32(0), jnp.int32(HI_BITS)))

        gt = keys_all > tau
        cgt = jnp.sum(gt.astype(jnp.int32))
        need = k - cgt
        sum_gt = jnp.sum(jnp.where(gt, nll_all, 0.0))

        tie = keys_all == tau
        tie_total = jnp.sum(tie.astype(jnp.int32))
        nll_tie = jnp.where(tie, nll_all, 0.0)

        def _tie_all():
            return jnp.sum(nll_tie)

        def _tie_search():
            idx = (lax.broadcasted_iota(jnp.int32, (NB, BLK), 0) * BLK
                   + lax.broadcasted_iota(jnp.int32, (NB, BLK), 1))

            def bs2(_, c):
                lo, hi = c
                mid = lo + (hi - lo + 1) // 2
                cnt = jnp.sum((tie & (idx < mid)).astype(jnp.int32))
                ok = cnt <= need
                return jnp.where(ok, mid, lo), jnp.where(ok, hi, mid - 1)

            mcut, _ = lax.fori_loop(0, 15, bs2, (jnp.int32(0), jnp.int32(N)))
            return jnp.sum(jnp.where(idx < mcut, nll_tie, 0.0))

        sum_tie = lax.cond(tie_total == need, _tie_all, _tie_search)

        total = accf[0] + sum_gt + sum_tie
        cnt_all = acci[0] + k
        loss_ref[:, :] = jnp.broadcast_to(total / cnt_all.astype(jnp.float32),
                                          (1, 1))


@jax.jit
def _run(outputs, labels3, gpad):
    return pl.pallas_call(
        _body,
        grid=(NB,),
        in_specs=[
            pl.BlockSpec((BLK, D), lambda i: (i, 0)),
            pl.BlockSpec((1, 1, BLK), lambda i: (i, 0, 0)),
            pl.BlockSpec((D, D), lambda i: (0, 0)),
        ],
        out_specs=pl.BlockSpec((1, 1), lambda i: (0, 0)),
        out_shape=jax.ShapeDtypeStruct((1, 1), jnp.float32),
        scratch_shapes=[
            pltpu.VMEM((NB, BLK), jnp.int32),
            pltpu.VMEM((NB, BLK), jnp.float32),
            pltpu.SMEM((1,), jnp.float32),
            pltpu.SMEM((2,), jnp.int32),
        ],
        compiler_params=pltpu.CompilerParams(
            dimension_semantics=("arbitrary",),
        ),
    )(outputs, labels3, gpad)


def kernel(outputs, labels, global_logit):
    outputs = outputs.astype(jnp.float32)
    labels3 = labels.astype(jnp.int32).reshape(NB, 1, BLK)
    gpad = jnp.zeros((D, D), jnp.float32).at[:G].set(global_logit)
    return _run(outputs, labels3, gpad)[0, 0]


# speedup vs baseline: 1.4161x; 1.0119x over previous
import jax
import jax.numpy as jnp
from jax import lax
from jax.experimental import pallas as pl
from jax.experimental.pallas import tpu as pltpu

N = 16384
D = 128
G = 100
BLK = 4096
NB = N // BLK
EPS = 1e-8
NEG = -1e30
HI_BITS = 0x40000001


def _body(out_ref, lbl_ref, g_ref, loss_ref, keys_s, nll_s, accf, acci):
    i = pl.program_id(0)

    @pl.when(i == 0)
    def _init():
        accf[0] = 0.0
        acci[0] = 0
        acci[1] = 0

    x = out_ref[:, :]
    g = g_ref[:, :]

    row = lax.broadcasted_iota(jnp.int32, (D, BLK), 0)

    gss = jnp.sum(g * g, axis=1, keepdims=True)
    gn = g / jnp.maximum(jnp.sqrt(gss), EPS)

    ones_r = jnp.ones((1, D), jnp.float32)
    dn = (((1,), (1,)), ((), ()))
    dn0 = (((1,), (0,)), ((), ()))
    xT = jnp.transpose(x)
    simsT = lax.dot_general(gn, x, dn, preferred_element_type=jnp.float32)

    xss = lax.dot_general(ones_r, xT * xT, dn0,
                          preferred_element_type=jnp.float32)
    nrm = jnp.maximum(jnp.sqrt(xss), EPS)
    rinv = 1.0 / nrm
    simsm = jnp.where(row < G, simsT * rinv, NEG)

    top1 = jnp.max(simsm, axis=0, keepdims=True)
    arg1 = jnp.min(jnp.where(simsm == top1, row, D), axis=0, keepdims=True)
    top2 = jnp.max(jnp.where(row == arg1, NEG, simsm), axis=0, keepdims=True)
    diff = top1 - top2

    esum = lax.dot_general(ones_r, jnp.exp(xT - nrm), dn0,
                           preferred_element_type=jnp.float32)
    lse = nrm + jnp.log(esum)

    lbl = lbl_ref[0].astype(jnp.int32)
    p_mask = lbl <= G - 1
    tgt = jnp.where(p_mask, lbl, arg1)
    tval = lax.dot_general(ones_r, jnp.where(row == tgt, xT, 0.0), dn0,
                           preferred_element_type=jnp.float32)
    nll = lse - tval

    u_mask = jnp.logical_not(p_mask)
    keys = jnp.where(u_mask, lax.bitcast_convert_type(diff, jnp.int32),
                     jnp.int32(-1))

    keys_s[pl.ds(i, 1), :] = keys
    nll_s[pl.ds(i, 1), :] = jnp.where(u_mask, nll, 0.0)

    accf[0] += jnp.sum(jnp.where(p_mask, nll, 0.0))
    acci[0] += jnp.sum(p_mask.astype(jnp.int32))
    acci[1] += jnp.sum(u_mask.astype(jnp.int32))

    @pl.when(i == NB - 1)
    def _finish():
        k = acci[1] // 10
        keys_all = keys_s[:, :]
        nll_all = nll_s[:, :]

        def bs(_, c):
            lo, hi = c
            qq = jnp.maximum((hi - lo) // 4, 1)
            m1 = lo + qq
            m2 = lo + 2 * qq
            m3 = lo + 3 * qq
            c1 = jnp.sum((keys_all >= m1).astype(jnp.int32))
            c2 = jnp.sum((keys_all >= m2).astype(jnp.int32))
            c3 = jnp.sum((keys_all >= m3).astype(jnp.int32))
            ok1 = c1 >= k
            ok2 = c2 >= k
            ok3 = c3 >= k
            lo_n = jnp.where(ok3, m3, jnp.where(ok2, m2, jnp.where(ok1, m1, lo)))
            hi_n = jnp.where(ok3, hi, jnp.where(ok2, m3, jnp.where(ok1, m2, m1)))
            return lo_n, hi_n

        tau, _ = lax.fori_loop(0, 18, bs,
                               (jnp.int32(0), jnp.int32(HI_BITS)))

        gt = keys_all > tau
        cgt = jnp.sum(gt.astype(jnp.int32))
        need = k - cgt
        sum_gt = jnp.sum(jnp.where(gt, nll_all, 0.0))

        tie = keys_all == tau
        tie_total = jnp.sum(tie.astype(jnp.int32))
        nll_tie = jnp.where(tie, nll_all, 0.0)

        def _tie_all():
            return jnp.sum(nll_tie)

        def _tie_search():
            idx = (lax.broadcasted_iota(jnp.int32, (NB, BLK), 0) * BLK
                   + lax.broadcasted_iota(jnp.int32, (NB, BLK), 1))

            def bs2(_, c):
                lo, hi = c
                mid = lo + (hi - lo + 1) // 2
                cnt = jnp.sum((tie & (idx < mid)).astype(jnp.int32))
                ok = cnt <= need
                return jnp.where(ok, mid, lo), jnp.where(ok, hi, mid - 1)

            mcut, _ = lax.fori_loop(0, 15, bs2, (jnp.int32(0), jnp.int32(N)))
            return jnp.sum(jnp.where(idx < mcut, nll_tie, 0.0))

        sum_tie = lax.cond(tie_total == need, _tie_all, _tie_search)

        total = accf[0] + sum_gt + sum_tie
        cnt_all = acci[0] + k
        loss_ref[:, :] = jnp.broadcast_to(total / cnt_all.astype(jnp.float32),
                                          (1, 1))


@jax.jit
def _run(outputs, labels3, gpad):
    return pl.pallas_call(
        _body,
        grid=(NB,),
        in_specs=[
            pl.BlockSpec((BLK, D), lambda i: (i, 0)),
            pl.BlockSpec((1, 1, BLK), lambda i: (i, 0, 0)),
            pl.BlockSpec((D, D), lambda i: (0, 0)),
        ],
        out_specs=pl.BlockSpec((1, 1), lambda i: (0, 0)),
        out_shape=jax.ShapeDtypeStruct((1, 1), jnp.float32),
        scratch_shapes=[
            pltpu.VMEM((NB, BLK), jnp.int32),
            pltpu.VMEM((NB, BLK), jnp.float32),
            pltpu.SMEM((1,), jnp.float32),
            pltpu.SMEM((2,), jnp.int32),
        ],
        compiler_params=pltpu.CompilerParams(
            dimension_semantics=("arbitrary",),
        ),
    )(outputs, labels3, gpad)


def kernel(outputs, labels, global_logit):
    outputs = outputs.astype(jnp.float32)
    labels3 = labels.astype(jnp.int32).reshape(NB, 1, BLK)
    gpad = jnp.zeros((D, D), jnp.float32).at[:G].set(global_logit)
    return _run(outputs, labels3, gpad)[0, 0]
